# baseline (speedup 1.0000x reference)
import functools

import jax
import jax.numpy as jnp
from jax import lax
from jax.experimental import pallas as pl
from jax.experimental.pallas import tpu as pltpu
from jax.experimental.pallas import tpu_sc as plsc

NUM_NODES = 100000
VOCAB = 119
EMB_DIM = 128

NC = 2
NS = 16
NW = NC * NS

CB = 3200
SUB = 400
NSUB = CB // SUB
NBUF = 2

TC_BLK = 1000


def _emb_body(idx_hbm, table_hbm, out_hbm, idx_v, rows_v, table_sh, gsems, ssems):
    sid = lax.axis_index("s")
    wid = sid * NC + lax.axis_index("c")
    base = pl.multiple_of(jnp.minimum(wid * CB, NUM_NODES - CB), 8)

    @pl.when(sid == 0)
    def _():
        pltpu.sync_copy(table_hbm, table_sh)

    pltpu.sync_copy(idx_hbm.at[pl.ds(base, CB)], idx_v)
    plsc.subcore_barrier()

    def gather(j, b):
        return pltpu.make_async_copy(
            table_sh.at[idx_v.at[pl.ds(j * SUB, SUB)]], rows_v.at[b], gsems.at[b]
        )

    def scatter(j, b):
        return pltpu.make_async_copy(
            rows_v.at[b], out_hbm.at[pl.ds(base + j * SUB, SUB)], ssems.at[b]
        )

    gather(0, 0).start()
    for j in range(NSUB):
        b = j % NBUF
        gather(j, b).wait()
        if j + 1 < NSUB:
            nb = (j + 1) % NBUF
            if j + 1 >= NBUF:
                scatter(j + 1 - NBUF, nb).wait()
            gather(j + 1, nb).start()
        scatter(j, b).start()
    for j in range(max(0, NSUB - NBUF), NSUB):
        scatter(j, j % NBUF).wait()


def _sc_lookup(atomic_numbers, emb_table):
    mesh = plsc.VectorSubcoreMesh(core_axis_name="c", subcore_axis_name="s")
    fn = functools.partial(
        pl.kernel,
        mesh=mesh,
        out_type=jax.ShapeDtypeStruct((NUM_NODES, EMB_DIM), jnp.float32),
        scratch_types=[
            pltpu.VMEM((CB,), jnp.int32),
            pltpu.VMEM((NBUF, SUB, EMB_DIM), jnp.float32),
            pltpu.VMEM_SHARED((VOCAB, EMB_DIM), jnp.float32),
            pltpu.SemaphoreType.DMA((NBUF,)),
            pltpu.SemaphoreType.DMA((NBUF,)),
        ],
    )(_emb_body)
    return fn(atomic_numbers, emb_table)


def _tc_body(idx_ref, table_ref, out_ref):
    idx = idx_ref[0, 0, :]
    pos = jax.lax.broadcasted_iota(jnp.int32, (TC_BLK, 128), 1)
    one_hot = (idx[:, None] == pos).astype(jnp.float32)
    out_ref[...] = jnp.dot(one_hot, table_ref[...],
                           preferred_element_type=jnp.float32)


def _tc_lookup(atomic_numbers, table_pad):
    idx3 = atomic_numbers.reshape(NUM_NODES // TC_BLK, 1, TC_BLK)
    return pl.pallas_call(
        _tc_body,
        grid=(NUM_NODES // TC_BLK,),
        in_specs=[
            pl.BlockSpec((1, 1, TC_BLK), lambda i: (i, 0, 0)),
            pl.BlockSpec((128, 128), lambda i: (0, 0)),
        ],
        out_specs=pl.BlockSpec((TC_BLK, EMB_DIM), lambda i: (i, 0)),
        out_shape=jax.ShapeDtypeStruct((NUM_NODES, EMB_DIM), jnp.float32),
    )(idx3, table_pad)


def kernel(atomic_numbers, emb_table):
    idx = atomic_numbers.astype(jnp.int32)
    node_attrs = _sc_lookup(idx, emb_table)
    table_pad = jnp.pad(emb_table, ((0, 128 - VOCAB), (0, 0)))
    node_features = _tc_lookup(idx, table_pad)
    return (node_attrs, node_features)


# speedup vs baseline: 1.6291x; 1.6291x over previous
import functools

import jax
import jax.numpy as jnp
from jax import lax
from jax.experimental import pallas as pl
from jax.experimental.pallas import tpu as pltpu
from jax.experimental.pallas import tpu_sc as plsc

NUM_NODES = 100000
VOCAB = 119
EMB_DIM = 128

NC = 2
NS = 16
NW = NC * NS

CB = 3200
SUB = 400
NSUB = CB // SUB
NBUF = 2


def _emb_body(idx_hbm, table_hbm, out1_hbm, out2_hbm, idx_v, rows_v, table_sh,
              gsems, ssems):
    sid = lax.axis_index("s")
    wid = sid * NC + lax.axis_index("c")
    base = pl.multiple_of(jnp.minimum(wid * CB, NUM_NODES - CB), 8)

    @pl.when(sid == 0)
    def _():
        pltpu.sync_copy(table_hbm, table_sh)

    pltpu.sync_copy(idx_hbm.at[pl.ds(base, CB)], idx_v)
    plsc.subcore_barrier()

    def gather(j, b):
        return pltpu.make_async_copy(
            table_sh.at[idx_v.at[pl.ds(j * SUB, SUB)]], rows_v.at[b], gsems.at[b]
        )

    def scatters(j, b):
        return [
            pltpu.make_async_copy(
                rows_v.at[b], out.at[pl.ds(base + j * SUB, SUB)], ssems.at[b]
            )
            for out in (out1_hbm, out2_hbm)
        ]

    gather(0, 0).start()
    for j in range(NSUB):
        b = j % NBUF
        gather(j, b).wait()
        if j + 1 < NSUB:
            nb = (j + 1) % NBUF
            if j + 1 >= NBUF:
                for cp in scatters(j + 1 - NBUF, nb):
                    cp.wait()
            gather(j + 1, nb).start()
        for cp in scatters(j, b):
            cp.start()
    for j in range(max(0, NSUB - NBUF), NSUB):
        for cp in scatters(j, j % NBUF):
            cp.wait()


@functools.partial(jax.jit, static_argnums=())
def _emb_lookup(atomic_numbers, emb_table):
    mesh = plsc.VectorSubcoreMesh(core_axis_name="c", subcore_axis_name="s")
    fn = functools.partial(
        pl.kernel,
        mesh=mesh,
        out_type=(
            jax.ShapeDtypeStruct((NUM_NODES, EMB_DIM), jnp.float32),
            jax.ShapeDtypeStruct((NUM_NODES, EMB_DIM), jnp.float32),
        ),
        scratch_types=[
            pltpu.VMEM((CB,), jnp.int32),
            pltpu.VMEM((NBUF, SUB, EMB_DIM), jnp.float32),
            pltpu.VMEM_SHARED((VOCAB, EMB_DIM), jnp.float32),
            pltpu.SemaphoreType.DMA((NBUF,)),
            pltpu.SemaphoreType.DMA((NBUF,)),
        ],
    )(_emb_body)
    return fn(atomic_numbers, emb_table)


def kernel(atomic_numbers, emb_table):
    out1, out2 = _emb_lookup(atomic_numbers.astype(jnp.int32), emb_table)
    return (out1, out2)


# speedup vs baseline: 1.6338x; 1.0029x over previous
import functools

import jax
import jax.numpy as jnp
from jax import lax
from jax.experimental import pallas as pl
from jax.experimental.pallas import tpu as pltpu
from jax.experimental.pallas import tpu_sc as plsc

NUM_NODES = 100000
VOCAB = 119
EMB_DIM = 128

NC = 2
NS = 16
NW = NC * NS

CB = 3200
SUB = 320
NSUB = CB // SUB
NBUF = 3


def _emb_body(idx_hbm, table_hbm, out1_hbm, out2_hbm, idx_v, rows_v, table_sh,
              gsems, ssems):
    sid = lax.axis_index("s")
    wid = sid * NC + lax.axis_index("c")
    base = pl.multiple_of(jnp.minimum(wid * CB, NUM_NODES - CB), 8)

    @pl.when(sid == 0)
    def _():
        pltpu.sync_copy(table_hbm, table_sh)

    pltpu.sync_copy(idx_hbm.at[pl.ds(base, CB)], idx_v)
    plsc.subcore_barrier()

    def gather(j, b):
        return pltpu.make_async_copy(
            table_sh.at[idx_v.at[pl.ds(j * SUB, SUB)]], rows_v.at[b], gsems.at[b]
        )

    def scatters(j, b):
        return [
            pltpu.make_async_copy(
                rows_v.at[b], out.at[pl.ds(base + j * SUB, SUB)], ssems.at[b]
            )
            for out in (out1_hbm, out2_hbm)
        ]

    gather(0, 0).start()
    for j in range(NSUB):
        b = j % NBUF
        gather(j, b).wait()
        if j + 1 < NSUB:
            nb = (j + 1) % NBUF
            if j + 1 >= NBUF:
                for cp in scatters(j + 1 - NBUF, nb):
                    cp.wait()
            gather(j + 1, nb).start()
        for cp in scatters(j, b):
            cp.start()
    for j in range(max(0, NSUB - NBUF), NSUB):
        for cp in scatters(j, j % NBUF):
            cp.wait()


@functools.partial(jax.jit, static_argnums=())
def _emb_lookup(atomic_numbers, emb_table):
    mesh = plsc.VectorSubcoreMesh(core_axis_name="c", subcore_axis_name="s")
    fn = functools.partial(
        pl.kernel,
        mesh=mesh,
        out_type=(
            jax.ShapeDtypeStruct((NUM_NODES, EMB_DIM), jnp.float32),
            jax.ShapeDtypeStruct((NUM_NODES, EMB_DIM), jnp.float32),
        ),
        scratch_types=[
            pltpu.VMEM((CB,), jnp.int32),
            pltpu.VMEM((NBUF, SUB, EMB_DIM), jnp.float32),
            pltpu.VMEM_SHARED((VOCAB, EMB_DIM), jnp.float32),
            pltpu.SemaphoreType.DMA((NBUF,)),
            pltpu.SemaphoreType.DMA((NBUF,)),
        ],
    )(_emb_body)
    return fn(atomic_numbers, emb_table)


def kernel(atomic_numbers, emb_table):
    out1, out2 = _emb_lookup(atomic_numbers.astype(jnp.int32), emb_table)
    return (out1, out2)
